# baseline (speedup 1.0000x reference)
import functools

import jax
import jax.numpy as jnp
from jax import lax
from jax.experimental import pallas as pl
from jax.experimental.pallas import tpu as pltpu
from jax.experimental.pallas import tpu_sc as plsc

MAX_REL = 128
DIM = 64
ROWS = 2 * MAX_REL + 1
SEQ = 2048
NC = 2
NS = 16
NW = NC * NS
KB = SEQ // (NW // 2)
QC = SEQ // 2
NR = KB + QC - 1
NB = 16


def _sc_run(position_embeddings):
    mesh = plsc.VectorSubcoreMesh(core_axis_name="c", subcore_axis_name="s")

    @functools.partial(
        pl.kernel,
        mesh=mesh,
        out_type=jax.ShapeDtypeStruct((SEQ * SEQ * DIM,), jnp.float32),
        scratch_types=[
            pltpu.VMEM((ROWS, DIM), jnp.float32),
            pltpu.VMEM((NR * DIM,), jnp.float32),
            pltpu.SemaphoreType.DMA,
        ],
        compiler_params=pltpu.CompilerParams(use_tc_tiling_on_sc=False),
    )
    def run(table_hbm, out_hbm, table_v, bloc_v, sem):
        wid = lax.axis_index("s") * NC + lax.axis_index("c")
        k0 = (wid // 2) * KB
        q0 = (wid % 2) * QC

        pltpu.sync_copy(table_hbm, table_v)

        A = k0 - q0 - 1

        def build(j, carry):
            idx = jnp.clip(j - A, 0, ROWS - 1)
            base = j * DIM
            for c in range(DIM // 16):
                bloc_v[pl.ds(base + 16 * c, 16)] = (
                    table_v[idx, pl.ds(16 * c, 16)])
            return carry

        lax.fori_loop(0, NR, build, 0, unroll=False)

        def fire(i):
            pltpu.async_copy(
                bloc_v.at[pl.ds((KB - 1 - i) * DIM, QC * DIM)],
                out_hbm.at[pl.ds(((k0 + i) * SEQ + q0) * DIM, QC * DIM)],
                sem,
            )

        def wait_one():
            pltpu.make_async_copy(
                bloc_v.at[pl.ds(0, QC * DIM)],
                out_hbm.at[pl.ds((k0 * SEQ + q0) * DIM, QC * DIM)],
                sem,
            ).wait()

        for b in range(NB):
            fire(b)

        def roll(i, carry):
            wait_one()
            fire(i + NB)
            return carry

        lax.fori_loop(0, KB - NB, roll, 0, unroll=False)
        for b in range(NB):
            wait_one()

    return run(position_embeddings)


def kernel(query_length, key_length, position_embeddings):
    del query_length, key_length
    flat = _sc_run(position_embeddings)
    return flat.reshape(SEQ, SEQ, DIM)


# speedup vs baseline: 1.2606x; 1.2606x over previous
import functools

import jax
import jax.numpy as jnp
from jax import lax
from jax.experimental import pallas as pl
from jax.experimental.pallas import tpu as pltpu
from jax.experimental.pallas import tpu_sc as plsc

MAX_REL = 128
DIM = 64
ROWS = 2 * MAX_REL + 1
SEQ = 2048
NC = 2
NS = 16
NW = NC * NS
NPH = 8
BJ = 4104
JC = BJ // 4
BK = 8


def _sc_build_phases(position_embeddings):
    mesh = plsc.VectorSubcoreMesh(core_axis_name="c", subcore_axis_name="s")

    @functools.partial(
        pl.kernel,
        mesh=mesh,
        out_type=jax.ShapeDtypeStruct((NPH * BJ * DIM,), jnp.float32),
        scratch_types=[
            pltpu.VMEM((ROWS, DIM), jnp.float32),
            pltpu.VMEM((JC * DIM,), jnp.float32),
        ],
        compiler_params=pltpu.CompilerParams(use_tc_tiling_on_sc=False),
    )
    def run(table_hbm, out_hbm, table_v, bloc_v):
        wid = lax.axis_index("s") * NC + lax.axis_index("c")
        p = wid % NPH
        j0 = (wid // NPH) * JC

        pltpu.sync_copy(table_hbm, table_v)

        off = j0 + p - 1919

        def build(j, carry):
            idx = jnp.clip(j + off, 0, ROWS - 1)
            base = j * DIM
            for c in range(DIM // 16):
                bloc_v[pl.ds(base + 16 * c, 16)] = (
                    table_v[idx, pl.ds(16 * c, 16)])
            return carry

        lax.fori_loop(0, JC, build, 0, unroll=False)
        pltpu.sync_copy(
            bloc_v,
            out_hbm.at[pl.ds((p * BJ + j0) * DIM, JC * DIM)],
        )

    return run(position_embeddings).reshape(NPH, BJ, DIM)


def _tc_materialize(b_ph):
    def body(b_ref, out_ref):
        ib = pl.program_id(0)
        base = pl.multiple_of(BK * (SEQ // BK - 1 - ib), BK)
        for i in range(BK):
            out_ref[i] = b_ref[BK - 1 - i, pl.ds(base, SEQ), :]

    return pl.pallas_call(
        body,
        grid=(SEQ // BK,),
        in_specs=[pl.BlockSpec((NPH, BJ, DIM), lambda ib: (0, 0, 0))],
        out_specs=pl.BlockSpec((BK, SEQ, DIM), lambda ib: (ib, 0, 0)),
        out_shape=jax.ShapeDtypeStruct((SEQ, SEQ, DIM), jnp.float32),
    )(b_ph)


def kernel(query_length, key_length, position_embeddings):
    del query_length, key_length
    return _tc_materialize(_sc_build_phases(position_embeddings))


# speedup vs baseline: 4.2713x; 3.3884x over previous
import functools

import jax
import jax.numpy as jnp
from jax import lax
from jax.experimental import pallas as pl
from jax.experimental.pallas import tpu as pltpu
from jax.experimental.pallas import tpu_sc as plsc

MAX_REL = 128
DIM = 64
ROWS = 2 * MAX_REL + 1
SEQ = 2048
NC = 2
NS = 16
NW = NC * NS
BJ = 4352
JC = BJ // NW
BK = 8
QCH = 256
WW = 512


def _sc_build_bt(position_embeddings):
    mesh = plsc.VectorSubcoreMesh(core_axis_name="c", subcore_axis_name="s")

    @functools.partial(
        pl.kernel,
        mesh=mesh,
        out_type=jax.ShapeDtypeStruct((DIM, BJ), jnp.float32),
        scratch_types=[
            pltpu.VMEM((ROWS, DIM), jnp.float32),
            pltpu.VMEM((DIM, JC), jnp.float32),
        ],
        compiler_params=pltpu.CompilerParams(
            use_tc_tiling_on_sc=False, needs_layout_passes=False),
    )
    def run(table_hbm, out_hbm, table_v, bt_v):
        wid = lax.axis_index("s") * NC + lax.axis_index("c")
        j0 = wid * JC

        pltpu.sync_copy(table_hbm, table_v)

        lanes = lax.iota(jnp.int32, 16)

        def build(j, carry):
            idx = jnp.clip(j0 + j - 1919, 0, ROWS - 1)
            jvec = jnp.full((16,), 0, jnp.int32) + j
            for c in range(DIM // 16):
                v = table_v[idx, pl.ds(16 * c, 16)]
                plsc.store_scatter(bt_v, [16 * c + lanes, jvec], v)
            return carry

        lax.fori_loop(0, JC, build, 0, unroll=False)

        pltpu.sync_copy(bt_v, out_hbm.at[:, pl.ds(j0, JC)])

    return run(position_embeddings)


def _tc_materialize(b_t):
    def body(bt_ref, out_ref):
        ib = pl.program_id(0)
        base_last = SEQ - 1 - (ib * BK + (BK - 1))
        aligned = pl.multiple_of((base_last // 128) * 128, 128)
        for ch in range(SEQ // QCH):
            w = bt_ref[:, pl.ds(aligned + QCH * ch, WW)]
            for i in range(BK):
                ph = (SEQ - 1 - (ib * BK + i)) - aligned
                rolled = pltpu.roll(w, -ph, 1)
                out_ref[i, :, pl.ds(QCH * ch, QCH)] = rolled[:, :QCH]

    return pl.pallas_call(
        body,
        grid=(SEQ // BK,),
        in_specs=[pl.BlockSpec((DIM, BJ), lambda ib: (0, 0))],
        out_specs=pl.BlockSpec((BK, DIM, SEQ), lambda ib: (ib, 0, 0)),
        out_shape=jax.ShapeDtypeStruct((SEQ, DIM, SEQ), jnp.float32),
    )(b_t)


def kernel(query_length, key_length, position_embeddings):
    del query_length, key_length
    p = _tc_materialize(_sc_build_bt(position_embeddings))
    return p.transpose(0, 2, 1)
